# Initial kernel scaffold; baseline (speedup 1.0000x reference)
import jax
import jax.numpy as jnp
from jax.experimental import pallas as pl


def kernel(x, edge_index, W1, b1, W2, b2, W3, b3, Wc, bc):
    raise NotImplementedError("write your pallas kernel here")


# speedup vs baseline: 72.1248x; 72.1248x over previous
import functools

import jax
import jax.numpy as jnp
from jax import lax
from jax.experimental import pallas as pl
from jax.experimental.pallas import tpu as pltpu
from jax.experimental.pallas import tpu_sc as plsc

NW = 32
L = 16


@functools.cache
def _deg_kernel(n: int, e: int):
    ept = e // NW
    mesh = plsc.VectorSubcoreMesh(core_axis_name="c", subcore_axis_name="s")

    @functools.partial(
        pl.kernel,
        out_type=jax.ShapeDtypeStruct((NW, n), jnp.float32),
        mesh=mesh,
        scratch_types=[
            pltpu.VMEM((n,), jnp.float32),
            pltpu.VMEM((ept,), jnp.int32),
        ],
        compiler_params=pltpu.CompilerParams(needs_layout_passes=False),
    )
    def deg(dst_hbm, out_hbm, acc_v, dst_v):
        wid = lax.axis_index("s") * 2 + lax.axis_index("c")
        pltpu.sync_copy(dst_hbm.at[pl.ds(wid * ept, ept)], dst_v)
        zeros = jnp.zeros((L,), jnp.float32)
        ones = jnp.ones((L,), jnp.float32)

        def zbody(i, c):
            acc_v[pl.ds(i * L, L)] = zeros
            return c

        lax.fori_loop(0, n // L, zbody, 0, unroll=4)

        def ebody(i, c):
            dv = dst_v[pl.ds(i * L, L)]
            plsc.addupdate_scatter(acc_v, [dv], ones)
            return c

        lax.fori_loop(0, ept // L, ebody, 0, unroll=4)
        pltpu.sync_copy(acc_v, out_hbm.at[wid])

    return deg


@functools.cache
def _agg_kernel(F: int, n: int, e: int):
    ept = e // NW
    fn = F * n
    mesh = plsc.VectorSubcoreMesh(core_axis_name="c", subcore_axis_name="s")

    @functools.partial(
        pl.kernel,
        out_type=jax.ShapeDtypeStruct((NW, fn), jnp.float32),
        mesh=mesh,
        scratch_types=[
            pltpu.VMEM((fn,), jnp.float32),
            pltpu.VMEM((fn,), jnp.float32),
            pltpu.VMEM((ept,), jnp.int32),
            pltpu.VMEM((ept,), jnp.int32),
        ],
        compiler_params=pltpu.CompilerParams(needs_layout_passes=False),
    )
    def agg(g_hbm, src_hbm, dst_hbm, out_hbm, g_v, s_v, src_v, dst_v):
        wid = lax.axis_index("s") * 2 + lax.axis_index("c")
        base = wid * ept
        pltpu.sync_copy(g_hbm, g_v)
        pltpu.sync_copy(src_hbm.at[pl.ds(base, ept)], src_v)
        pltpu.sync_copy(dst_hbm.at[pl.ds(base, ept)], dst_v)
        zeros = jnp.zeros((L,), jnp.float32)

        def zbody(i, c):
            s_v[pl.ds(i * L, L)] = zeros
            return c

        lax.fori_loop(0, fn // L, zbody, 0, unroll=4)

        def ebody(i, c):
            sv = src_v[pl.ds(i * L, L)]
            dv = dst_v[pl.ds(i * L, L)]
            for f in range(F):
                off = jnp.int32(f * n)
                vals = plsc.load_gather(g_v, [sv + off])
                plsc.addupdate_scatter(s_v, [dv + off], vals)
            return c

        lax.fori_loop(0, ept // L, ebody, 0, unroll=2)
        pltpu.sync_copy(s_v, out_hbm.at[wid])

    return agg


def _tc_first(xT, W1, degp):
    n = xT.shape[1]
    F = W1.shape[0]

    def body(x_ref, w_ref, dp_ref, dis_ref, g_ref):
        deg = 1.0 + jnp.sum(dp_ref[...], axis=0, keepdims=True)
        dis = lax.rsqrt(deg)
        h = lax.dot_general(w_ref[...], x_ref[...], (((1,), (0,)), ((), ())),
                            preferred_element_type=jnp.float32)
        dis_ref[...] = dis
        g_ref[...] = h * dis

    return pl.pallas_call(
        body,
        out_shape=(jax.ShapeDtypeStruct((1, n), jnp.float32),
                   jax.ShapeDtypeStruct((F, n), jnp.float32)),
    )(xT, W1, degp)


def _tc_mid(sp, g, dis, b, Wn):
    F, n = g.shape
    Fn = Wn.shape[0]

    def body(sp_ref, g_ref, dis_ref, b_ref, w_ref, gn_ref):
        s = jnp.sum(sp_ref[...], axis=0)
        dis = dis_ref[...]
        a = jnp.tanh(dis * (s + g_ref[...]) + b_ref[...])
        h = lax.dot_general(w_ref[...], a, (((1,), (0,)), ((), ())),
                            preferred_element_type=jnp.float32)
        gn_ref[...] = h * dis

    return pl.pallas_call(
        body,
        out_shape=jax.ShapeDtypeStruct((Fn, n), jnp.float32),
    )(sp, g, dis, b, Wn)


def _tc_last(sp, g, dis, b, Wc, bc):
    F, n = g.shape
    C = Wc.shape[0]

    def body(sp_ref, g_ref, dis_ref, b_ref, wc_ref, bc_ref, out_ref, h_ref):
        s = jnp.sum(sp_ref[...], axis=0)
        dis = dis_ref[...]
        a = jnp.tanh(dis * (s + g_ref[...]) + b_ref[...])
        h_ref[...] = a
        out_ref[...] = lax.dot_general(wc_ref[...], a, (((1,), (0,)), ((), ())),
                                       preferred_element_type=jnp.float32) + bc_ref[...]

    return pl.pallas_call(
        body,
        out_shape=(jax.ShapeDtypeStruct((C, n), jnp.float32),
                   jax.ShapeDtypeStruct((F, n), jnp.float32)),
    )(sp, g, dis, b, Wc, bc)


@jax.jit
def kernel(x, edge_index, W1, b1, W2, b2, W3, b3, Wc, bc):
    n, d = x.shape
    e = edge_index.shape[1]
    assert e % (NW * L) == 0 and n % L == 0

    src = edge_index[0]
    dst = edge_index[1]
    xT = x.T

    degp = _deg_kernel(n, e)(dst)
    dis, g1 = _tc_first(xT, W1, degp)

    F1, F2, F3 = W2.shape[1], W3.shape[1], Wc.shape[1]
    s1 = _agg_kernel(F1, n, e)(g1.reshape(-1), src, dst)
    g2 = _tc_mid(s1.reshape(NW, F1, n), g1, dis, b1.reshape(-1, 1), W2)
    s2 = _agg_kernel(F2, n, e)(g2.reshape(-1), src, dst)
    g3 = _tc_mid(s2.reshape(NW, F2, n), g2, dis, b2.reshape(-1, 1), W3)
    s3 = _agg_kernel(F3, n, e)(g3.reshape(-1), src, dst)
    outT, hT = _tc_last(s3.reshape(NW, F3, n), g3, dis, b3.reshape(-1, 1),
                        Wc, bc.reshape(-1, 1))
    return outT.T, hT.T


# speedup vs baseline: 86.5491x; 1.2000x over previous
import functools

import jax
import jax.numpy as jnp
from jax import lax
from jax.experimental import pallas as pl
from jax.experimental.pallas import tpu as pltpu
from jax.experimental.pallas import tpu_sc as plsc

NW = 32
L = 16


@functools.cache
def _deg_kernel(n: int, e: int):
    ept = e // NW
    mesh = plsc.VectorSubcoreMesh(core_axis_name="c", subcore_axis_name="s")

    @functools.partial(
        pl.kernel,
        out_type=jax.ShapeDtypeStruct((NW, n), jnp.float32),
        mesh=mesh,
        scratch_types=[
            pltpu.VMEM((n,), jnp.float32),
            pltpu.VMEM((ept,), jnp.int32),
        ],
        compiler_params=pltpu.CompilerParams(needs_layout_passes=False),
    )
    def deg(dst_hbm, out_hbm, acc_v, dst_v):
        wid = lax.axis_index("s") * 2 + lax.axis_index("c")
        pltpu.sync_copy(dst_hbm.at[pl.ds(wid * ept, ept)], dst_v)
        zeros = jnp.zeros((L,), jnp.float32)
        ones = jnp.ones((L,), jnp.float32)

        @plsc.parallel_loop(0, n // L, unroll=4)
        def _(i):
            acc_v[pl.ds(i * L, L)] = zeros

        @plsc.parallel_loop(0, ept // L, unroll=4)
        def _(i):
            dv = dst_v[pl.ds(i * L, L)]
            plsc.addupdate_scatter(acc_v, [dv], ones)

        pltpu.sync_copy(acc_v, out_hbm.at[wid])

    return deg


@functools.cache
def _agg_kernel(F: int, n: int, e: int):
    ept = e // NW
    fn = F * n
    mesh = plsc.VectorSubcoreMesh(core_axis_name="c", subcore_axis_name="s")

    @functools.partial(
        pl.kernel,
        out_type=jax.ShapeDtypeStruct((NW, fn), jnp.float32),
        mesh=mesh,
        scratch_types=[
            pltpu.VMEM((fn,), jnp.float32),
            pltpu.VMEM((fn,), jnp.float32),
            pltpu.VMEM((ept,), jnp.int32),
            pltpu.VMEM((ept,), jnp.int32),
        ],
        compiler_params=pltpu.CompilerParams(needs_layout_passes=False),
    )
    def agg(g_hbm, src_hbm, dst_hbm, out_hbm, g_v, s_v, src_v, dst_v):
        wid = lax.axis_index("s") * 2 + lax.axis_index("c")
        base = wid * ept
        pltpu.sync_copy(g_hbm, g_v)
        pltpu.sync_copy(src_hbm.at[pl.ds(base, ept)], src_v)
        pltpu.sync_copy(dst_hbm.at[pl.ds(base, ept)], dst_v)
        zeros = jnp.zeros((L,), jnp.float32)

        @plsc.parallel_loop(0, fn // L, unroll=4)
        def _(i):
            s_v[pl.ds(i * L, L)] = zeros

        @plsc.parallel_loop(0, ept // L, unroll=4)
        def _(i):
            sv = src_v[pl.ds(i * L, L)]
            dv = dst_v[pl.ds(i * L, L)]
            for f in range(F):
                off = jnp.int32(f * n)
                vals = plsc.load_gather(g_v, [sv + off])
                plsc.addupdate_scatter(s_v, [dv + off], vals)

        pltpu.sync_copy(s_v, out_hbm.at[wid])

    return agg


def _tc_first(xT, W1, degp):
    n = xT.shape[1]
    F = W1.shape[0]

    def body(x_ref, w_ref, dp_ref, dis_ref, g_ref):
        deg = 1.0 + jnp.sum(dp_ref[...], axis=0, keepdims=True)
        dis = lax.rsqrt(deg)
        h = lax.dot_general(w_ref[...], x_ref[...], (((1,), (0,)), ((), ())),
                            preferred_element_type=jnp.float32)
        dis_ref[...] = dis
        g_ref[...] = h * dis

    return pl.pallas_call(
        body,
        out_shape=(jax.ShapeDtypeStruct((1, n), jnp.float32),
                   jax.ShapeDtypeStruct((F, n), jnp.float32)),
    )(xT, W1, degp)


def _tc_mid(sp, g, dis, b, Wn):
    F, n = g.shape
    Fn = Wn.shape[0]

    def body(sp_ref, g_ref, dis_ref, b_ref, w_ref, gn_ref):
        s = jnp.sum(sp_ref[...], axis=0)
        dis = dis_ref[...]
        a = jnp.tanh(dis * (s + g_ref[...]) + b_ref[...])
        h = lax.dot_general(w_ref[...], a, (((1,), (0,)), ((), ())),
                            preferred_element_type=jnp.float32)
        gn_ref[...] = h * dis

    return pl.pallas_call(
        body,
        out_shape=jax.ShapeDtypeStruct((Fn, n), jnp.float32),
    )(sp, g, dis, b, Wn)


def _tc_last(sp, g, dis, b, Wc, bc):
    F, n = g.shape
    C = Wc.shape[0]

    def body(sp_ref, g_ref, dis_ref, b_ref, wc_ref, bc_ref, out_ref, h_ref):
        s = jnp.sum(sp_ref[...], axis=0)
        dis = dis_ref[...]
        a = jnp.tanh(dis * (s + g_ref[...]) + b_ref[...])
        h_ref[...] = a
        out_ref[...] = lax.dot_general(wc_ref[...], a, (((1,), (0,)), ((), ())),
                                       preferred_element_type=jnp.float32) + bc_ref[...]

    return pl.pallas_call(
        body,
        out_shape=(jax.ShapeDtypeStruct((C, n), jnp.float32),
                   jax.ShapeDtypeStruct((F, n), jnp.float32)),
    )(sp, g, dis, b, Wc, bc)


@jax.jit
def kernel(x, edge_index, W1, b1, W2, b2, W3, b3, Wc, bc):
    n, d = x.shape
    e = edge_index.shape[1]
    assert e % (NW * L) == 0 and n % L == 0

    src = edge_index[0]
    dst = edge_index[1]
    xT = x.T

    degp = _deg_kernel(n, e)(dst)
    dis, g1 = _tc_first(xT, W1, degp)

    F1, F2, F3 = W2.shape[1], W3.shape[1], Wc.shape[1]
    s1 = _agg_kernel(F1, n, e)(g1.reshape(-1), src, dst)
    g2 = _tc_mid(s1.reshape(NW, F1, n), g1, dis, b1.reshape(-1, 1), W2)
    s2 = _agg_kernel(F2, n, e)(g2.reshape(-1), src, dst)
    g3 = _tc_mid(s2.reshape(NW, F2, n), g2, dis, b2.reshape(-1, 1), W3)
    s3 = _agg_kernel(F3, n, e)(g3.reshape(-1), src, dst)
    outT, hT = _tc_last(s3.reshape(NW, F3, n), g3, dis, b3.reshape(-1, 1),
                        Wc, bc.reshape(-1, 1))
    return outT.T, hT.T
